# baseline (speedup 1.0000x reference)
import functools

import jax
import jax.numpy as jnp
from jax import lax
from jax.experimental import pallas as pl
from jax.experimental.pallas import tpu as pltpu
from jax.experimental.pallas import tpu_sc as plsc

VOCAB = 1000000
D_EMBED = 64
BATCH = 16384
MAX_TOKEN_LEN = 20
B_TOTAL = BATCH * MAX_TOKEN_LEN

NUM_CORES = 2
NUM_SUBCORES = 16
NW = NUM_CORES * NUM_SUBCORES

BLK = 256
NBLK_UNIF = 122
LEFTOVER_BLK_BASE = 3904 * BLK
RAGGED_BASE = 3906 * BLK
L = 16

B_PER_W = B_TOTAL // NW
CHUNK = 128
NCH = B_PER_W // CHUNK
GROUP = 5
NGRP = NCH // GROUP
NCH_TOTAL = B_TOTAL // CHUNK


def _make_detile():
    mesh = plsc.VectorSubcoreMesh(core_axis_name="c", subcore_axis_name="s")

    @functools.partial(
        pl.kernel,
        mesh=mesh,
        out_type=jax.ShapeDtypeStruct((VOCAB * D_EMBED,), jnp.float32),
        scratch_types=[
            pltpu.VMEM((D_EMBED, BLK), jnp.float32),
            pltpu.VMEM((D_EMBED, BLK), jnp.float32),
            pltpu.VMEM((BLK * D_EMBED,), jnp.float32),
            pltpu.VMEM((BLK * D_EMBED,), jnp.float32),
            pltpu.VMEM((D_EMBED, 64), jnp.float32),
            pltpu.VMEM((64 * D_EMBED,), jnp.float32),
            pltpu.SemaphoreType.DMA,
            pltpu.SemaphoreType.DMA,
            pltpu.SemaphoreType.DMA,
            pltpu.SemaphoreType.DMA,
        ],
        compiler_params=pltpu.CompilerParams(
            use_tc_tiling_on_sc=True, needs_layout_passes=False
        ),
    )
    def detile_kernel(tbl_t, out, in_v0, in_v1, out_v0, out_v1, tin_v, tout_v,
                      si0, si1, so0, so1):
        wid = lax.axis_index("s") * NUM_CORES + lax.axis_index("c")
        in_v = (in_v0, in_v1)
        out_v = (out_v0, out_v1)
        sem_i = (si0, si1)
        sem_o = (so0, so1)

        iota = lax.iota(jnp.int32, L)
        rot = [(iota + c0) & (L - 1) for c0 in range(L)]

        def issue_in(base, par):
            pltpu.async_copy(
                tbl_t.at[:, pl.ds(pl.multiple_of(base, BLK), BLK)],
                in_v[par],
                sem_i[par],
            )

        def wait_in(par):
            pltpu.make_async_copy(
                tbl_t.at[:, pl.ds(0, BLK)], in_v[par], sem_i[par]
            ).wait()

        def issue_out(base, par):
            pltpu.async_copy(
                out_v[par],
                out.at[
                    pl.ds(
                        pl.multiple_of(base * D_EMBED, BLK * D_EMBED),
                        BLK * D_EMBED,
                    )
                ],
                sem_o[par],
            )

        def wait_out(par):
            pltpu.make_async_copy(
                out_v[par], out.at[pl.ds(0, BLK * D_EMBED)], sem_o[par]
            ).wait()

        def diag_transpose(in_ref, out_ref, nrows):
            @plsc.parallel_loop(0, nrows // L, 1, unroll=2)
            def _(rq):
                idx_r = iota + rq * L
                base_r = idx_r * D_EMBED
                for chi in range(0, D_EMBED, L):
                    for c0 in range(L):
                        idx_c = rot[c0] + chi if chi else rot[c0]
                        v = plsc.load_gather(in_ref, [idx_c, idx_r])
                        plsc.store_scatter(out_ref, [base_r + idx_c], v)

        def transpose_block(par):
            diag_transpose(in_v[par], out_v[par], BLK)

        def block_base(j):
            return (wid + j * NW) * BLK

        def step(j, par):
            wait_in(par)

            @pl.when(j >= 2)
            def _():
                wait_out(par)

            transpose_block(par)
            issue_out(block_base(j), par)

            @pl.when(j < NBLK_UNIF - 2)
            def _():
                issue_in(block_base(j + 2), par)

        issue_in(block_base(0), 0)
        issue_in(block_base(1), 1)

        def body(u, carry):
            step(2 * u, 0)
            step(2 * u + 1, 1)
            return carry

        lax.fori_loop(0, NBLK_UNIF // 2, body, 0)
        wait_out(0)
        wait_out(1)

        def serial_block(base):
            issue_in(base, 0)
            wait_in(0)
            transpose_block(0)
            issue_out(base, 0)
            wait_out(0)

        @pl.when(wid < 2)
        def _():
            serial_block(LEFTOVER_BLK_BASE + wid * BLK)

        @pl.when(wid == 2)
        def _():
            pltpu.sync_copy(tbl_t.at[:, pl.ds(RAGGED_BASE, 64)], tin_v)
            diag_transpose(tin_v, tout_v, 64)
            pltpu.sync_copy(
                tout_v, out.at[pl.ds(RAGGED_BASE * D_EMBED, 64 * D_EMBED)]
            )

    return detile_kernel


def _make_gather():
    mesh = plsc.VectorSubcoreMesh(core_axis_name="c", subcore_axis_name="s")

    @functools.partial(
        pl.kernel,
        mesh=mesh,
        out_type=jax.ShapeDtypeStruct((NCH_TOTAL, CHUNK, D_EMBED), jnp.float32),
        scratch_types=[
            pltpu.VMEM((NCH, CHUNK), jnp.int32),
            pltpu.VMEM((2, GROUP, CHUNK, D_EMBED), jnp.float32),
            pltpu.SemaphoreType.DMA,
            pltpu.SemaphoreType.DMA,
            pltpu.SemaphoreType.DMA,
            pltpu.SemaphoreType.DMA,
        ],
        compiler_params=pltpu.CompilerParams(use_tc_tiling_on_sc=False),
    )
    def gather_kernel(idx_hbm, table_hbm, out_hbm, idx_v, rows_v,
                      sem_g0, sem_g1, sem_w0, sem_w1):
        wid = lax.axis_index("s") * NUM_CORES + lax.axis_index("c")
        grp_base = wid * NGRP
        sem_g = (sem_g0, sem_g1)
        sem_w = (sem_w0, sem_w1)

        pltpu.sync_copy(idx_hbm.at[wid], idx_v)

        def issue_group(g, parity):
            for b in range(GROUP):
                pltpu.async_copy(
                    table_hbm.at[idx_v.at[g * GROUP + b]],
                    rows_v.at[parity].at[b],
                    sem_g[parity],
                )

        def wait_gathers(parity):
            pltpu.make_async_copy(
                out_hbm.at[pl.ds(0, GROUP)], rows_v.at[parity], sem_g[parity]
            ).wait()

        def out_copy(g, parity):
            pltpu.async_copy(
                rows_v.at[parity],
                out_hbm.at[pl.ds((grp_base + g) * GROUP, GROUP)],
                sem_w[parity],
            )

        def wait_out(parity):
            pltpu.make_async_copy(
                rows_v.at[parity], out_hbm.at[pl.ds(0, GROUP)], sem_w[parity]
            ).wait()

        issue_group(0, 0)
        issue_group(1, 1)
        wait_gathers(0)
        out_copy(0, 0)

        def step(t, parity_cur):
            wait_out(1 - parity_cur)
            issue_group(t + 1, 1 - parity_cur)
            wait_gathers(parity_cur)
            out_copy(t, parity_cur)

        def body(u, carry):
            step(2 * u + 1, 1)
            step(2 * u + 2, 0)
            return carry

        lax.fori_loop(0, (NGRP - 2) // 2, body, 0)

        wait_gathers(1)
        out_copy(NGRP - 1, 1)
        wait_out(0)
        wait_out(1)

    return gather_kernel


_detile = _make_detile()
_gather = _make_gather()


@jax.jit
def kernel(scenario_tag_ids, tag_embedding_weight):
    idx = scenario_tag_ids.reshape(NW, NCH, CHUNK).astype(jnp.int32)
    tbl_flat = _detile(tag_embedding_weight.T)
    tbl_lin = tbl_flat.reshape(VOCAB, D_EMBED)
    out = _gather(idx, tbl_lin)
    return out.reshape(BATCH, MAX_TOKEN_LEN, D_EMBED)


# speedup vs baseline: 1.0019x; 1.0019x over previous
import functools

import jax
import jax.numpy as jnp
from jax import lax
from jax.experimental import pallas as pl
from jax.experimental.pallas import tpu as pltpu
from jax.experimental.pallas import tpu_sc as plsc

VOCAB = 1000000
D_EMBED = 64
BATCH = 16384
MAX_TOKEN_LEN = 20
B_TOTAL = BATCH * MAX_TOKEN_LEN

NUM_CORES = 2
NUM_SUBCORES = 16
NW = NUM_CORES * NUM_SUBCORES

BLK = 256
NBLK_UNIF = 122
LEFTOVER_BLK_BASE = 3904 * BLK
RAGGED_BASE = 3906 * BLK
L = 16

B_PER_W = B_TOTAL // NW
CHUNK = 128
NCH = B_PER_W // CHUNK
GROUP = 4
NGRP = NCH // GROUP
NCH_TOTAL = B_TOTAL // CHUNK


def _make_detile():
    mesh = plsc.VectorSubcoreMesh(core_axis_name="c", subcore_axis_name="s")

    @functools.partial(
        pl.kernel,
        mesh=mesh,
        out_type=jax.ShapeDtypeStruct((VOCAB * D_EMBED,), jnp.float32),
        scratch_types=[
            pltpu.VMEM((D_EMBED, BLK), jnp.float32),
            pltpu.VMEM((D_EMBED, BLK), jnp.float32),
            pltpu.VMEM((BLK * D_EMBED,), jnp.float32),
            pltpu.VMEM((BLK * D_EMBED,), jnp.float32),
            pltpu.VMEM((D_EMBED, 64), jnp.float32),
            pltpu.VMEM((64 * D_EMBED,), jnp.float32),
            pltpu.SemaphoreType.DMA,
            pltpu.SemaphoreType.DMA,
            pltpu.SemaphoreType.DMA,
            pltpu.SemaphoreType.DMA,
        ],
        compiler_params=pltpu.CompilerParams(
            use_tc_tiling_on_sc=True, needs_layout_passes=False
        ),
    )
    def detile_kernel(tbl_t, out, in_v0, in_v1, out_v0, out_v1, tin_v, tout_v,
                      si0, si1, so0, so1):
        wid = lax.axis_index("s") * NUM_CORES + lax.axis_index("c")
        in_v = (in_v0, in_v1)
        out_v = (out_v0, out_v1)
        sem_i = (si0, si1)
        sem_o = (so0, so1)

        iota = lax.iota(jnp.int32, L)
        rot = [(iota + c0) & (L - 1) for c0 in range(L)]

        def issue_in(base, par):
            pltpu.async_copy(
                tbl_t.at[:, pl.ds(pl.multiple_of(base, BLK), BLK)],
                in_v[par],
                sem_i[par],
            )

        def wait_in(par):
            pltpu.make_async_copy(
                tbl_t.at[:, pl.ds(0, BLK)], in_v[par], sem_i[par]
            ).wait()

        def issue_out(base, par):
            pltpu.async_copy(
                out_v[par],
                out.at[
                    pl.ds(
                        pl.multiple_of(base * D_EMBED, BLK * D_EMBED),
                        BLK * D_EMBED,
                    )
                ],
                sem_o[par],
            )

        def wait_out(par):
            pltpu.make_async_copy(
                out_v[par], out.at[pl.ds(0, BLK * D_EMBED)], sem_o[par]
            ).wait()

        def diag_transpose(in_ref, out_ref, nrows):
            @plsc.parallel_loop(0, nrows // L, 1, unroll=2)
            def _(rq):
                idx_r = iota + rq * L
                base_r = idx_r * D_EMBED
                for chi in range(0, D_EMBED, L):
                    for c0 in range(L):
                        idx_c = rot[c0] + chi if chi else rot[c0]
                        v = plsc.load_gather(in_ref, [idx_c, idx_r])
                        plsc.store_scatter(out_ref, [base_r + idx_c], v)

        def transpose_block(par):
            diag_transpose(in_v[par], out_v[par], BLK)

        def block_base(j):
            return (wid + j * NW) * BLK

        def step(j, par):
            wait_in(par)

            @pl.when(j >= 2)
            def _():
                wait_out(par)

            transpose_block(par)
            issue_out(block_base(j), par)

            @pl.when(j < NBLK_UNIF - 2)
            def _():
                issue_in(block_base(j + 2), par)

        issue_in(block_base(0), 0)
        issue_in(block_base(1), 1)

        def body(u, carry):
            step(2 * u, 0)
            step(2 * u + 1, 1)
            return carry

        lax.fori_loop(0, NBLK_UNIF // 2, body, 0)
        wait_out(0)
        wait_out(1)

        def serial_block(base):
            issue_in(base, 0)
            wait_in(0)
            transpose_block(0)
            issue_out(base, 0)
            wait_out(0)

        @pl.when(wid < 2)
        def _():
            serial_block(LEFTOVER_BLK_BASE + wid * BLK)

        @pl.when(wid == 2)
        def _():
            pltpu.sync_copy(tbl_t.at[:, pl.ds(RAGGED_BASE, 64)], tin_v)
            diag_transpose(tin_v, tout_v, 64)
            pltpu.sync_copy(
                tout_v, out.at[pl.ds(RAGGED_BASE * D_EMBED, 64 * D_EMBED)]
            )

    return detile_kernel


def _make_gather():
    mesh = plsc.VectorSubcoreMesh(core_axis_name="c", subcore_axis_name="s")

    @functools.partial(
        pl.kernel,
        mesh=mesh,
        out_type=jax.ShapeDtypeStruct((NCH_TOTAL, CHUNK, D_EMBED), jnp.float32),
        scratch_types=[
            pltpu.VMEM((NCH, CHUNK), jnp.int32),
            pltpu.VMEM((2, GROUP, CHUNK, D_EMBED), jnp.float32),
            pltpu.SemaphoreType.DMA,
            pltpu.SemaphoreType.DMA,
            pltpu.SemaphoreType.DMA,
            pltpu.SemaphoreType.DMA,
        ],
        compiler_params=pltpu.CompilerParams(use_tc_tiling_on_sc=False),
    )
    def gather_kernel(idx_hbm, table_hbm, out_hbm, idx_v, rows_v,
                      sem_g0, sem_g1, sem_w0, sem_w1):
        wid = lax.axis_index("s") * NUM_CORES + lax.axis_index("c")
        grp_base = wid * NGRP
        sem_g = (sem_g0, sem_g1)
        sem_w = (sem_w0, sem_w1)

        pltpu.sync_copy(idx_hbm.at[wid], idx_v)

        def issue_group(g, parity):
            for b in range(GROUP):
                pltpu.async_copy(
                    table_hbm.at[idx_v.at[g * GROUP + b]],
                    rows_v.at[parity].at[b],
                    sem_g[parity],
                )

        def wait_gathers(parity):
            pltpu.make_async_copy(
                out_hbm.at[pl.ds(0, GROUP)], rows_v.at[parity], sem_g[parity]
            ).wait()

        def out_copy(g, parity):
            pltpu.async_copy(
                rows_v.at[parity],
                out_hbm.at[pl.ds((grp_base + g) * GROUP, GROUP)],
                sem_w[parity],
            )

        def wait_out(parity):
            pltpu.make_async_copy(
                rows_v.at[parity], out_hbm.at[pl.ds(0, GROUP)], sem_w[parity]
            ).wait()

        issue_group(0, 0)
        issue_group(1, 1)
        wait_gathers(0)
        out_copy(0, 0)

        def step(t, parity_cur):
            wait_out(1 - parity_cur)
            issue_group(t + 1, 1 - parity_cur)
            wait_gathers(parity_cur)
            out_copy(t, parity_cur)

        def body(u, carry):
            step(2 * u + 1, 1)
            step(2 * u + 2, 0)
            return carry

        lax.fori_loop(0, (NGRP - 2) // 2, body, 0)

        wait_gathers(1)
        out_copy(NGRP - 1, 1)
        wait_out(0)
        wait_out(1)

    return gather_kernel


_detile = _make_detile()
_gather = _make_gather()


@jax.jit
def kernel(scenario_tag_ids, tag_embedding_weight):
    idx = scenario_tag_ids.reshape(NW, NCH, CHUNK).astype(jnp.int32)
    tbl_flat = _detile(tag_embedding_weight.T)
    tbl_lin = tbl_flat.reshape(VOCAB, D_EMBED)
    out = _gather(idx, tbl_lin)
    return out.reshape(BATCH, MAX_TOKEN_LEN, D_EMBED)
